# baseline (speedup 1.0000x reference)
import functools

import jax
import jax.numpy as jnp
from jax import lax
from jax.experimental import pallas as pl
from jax.experimental.pallas import tpu as pltpu
from jax.experimental.pallas import tpu_sc as plsc

VOCAB = 1000000
DIM = 32
NVALS = 81920

_info = plsc.get_sparse_core_info()
_NC, _NS = _info.num_cores, _info.num_subcores
_NW = _NC * _NS
_BPW = NVALS // _NW


_mesh = plsc.VectorSubcoreMesh(core_axis_name="c", subcore_axis_name="s")


@functools.partial(
    pl.kernel,
    mesh=_mesh,
    out_type=[jax.ShapeDtypeStruct((NVALS, DIM), jnp.float32)] * 4,
    scratch_types=[
        pltpu.VMEM((_BPW,), jnp.int32),
        pltpu.VMEM((_BPW, DIM), jnp.float32),
        pltpu.SemaphoreType.DMA,
    ],
    compiler_params=pltpu.CompilerParams(use_tc_tiling_on_sc=False),
)
def _gather4(v1, v2, v3, v4, t1, t2, t3, t4, o1, o2, o3, o4,
             idx_v, rows_v, sem):
    wid = lax.axis_index("s") * _NC + lax.axis_index("c")
    base = wid * _BPW
    for vals, tab, out in ((v1, t1, o1), (v2, t2, o2),
                           (v3, t3, o3), (v4, t4, o4)):
        pltpu.sync_copy(vals.at[pl.ds(base, _BPW)], idx_v)
        pltpu.async_copy(tab.at[idx_v], rows_v, sem).wait()
        pltpu.sync_copy(rows_v, out.at[pl.ds(base, _BPW)])


def kernel(values_f1, lengths_f1, values_f2, lengths_f2,
           values_f3, lengths_f3, values_f4, lengths_f4,
           table_f1, table_f2, table_f3, table_f4):
    eye = jnp.eye(DIM, dtype=jnp.float32)
    o1, o2, o3, o4 = _gather4(values_f1, values_f2, values_f3, values_f4,
                              table_f1 @ eye, table_f2 @ eye,
                              table_f3 @ eye, table_f4 @ eye)
    return (o1, lengths_f1, o2, lengths_f2, o3, lengths_f3, o4, lengths_f4)


# speedup vs baseline: 1.3880x; 1.3880x over previous
import functools

import jax
import jax.numpy as jnp
from jax import lax
from jax.experimental import pallas as pl
from jax.experimental.pallas import tpu as pltpu
from jax.experimental.pallas import tpu_sc as plsc

VOCAB = 1000000
DIM = 32
NVALS = 81920
CDIM = 4 * DIM

_info = plsc.get_sparse_core_info()
_NC, _NS = _info.num_cores, _info.num_subcores
_NW = _NC * _NS
_BPW = NVALS // _NW
_CHUNK = 640
_NCHUNK = _BPW // _CHUNK


_mesh = plsc.VectorSubcoreMesh(core_axis_name="c", subcore_axis_name="s")


@functools.partial(
    pl.kernel,
    mesh=_mesh,
    out_type=[jax.ShapeDtypeStruct((NVALS, CDIM), jnp.float32)] * 4,
    scratch_types=[
        pltpu.VMEM((_BPW,), jnp.int32),
        pltpu.VMEM((_CHUNK, CDIM), jnp.float32),
        pltpu.SemaphoreType.DMA,
    ],
)
def _gather4(v1, v2, v3, v4, tab, o1, o2, o3, o4, idx_v, rows_v, sem):
    wid = lax.axis_index("s") * _NC + lax.axis_index("c")
    base = wid * _BPW
    for vals, out in ((v1, o1), (v2, o2), (v3, o3), (v4, o4)):
        pltpu.sync_copy(vals.at[pl.ds(base, _BPW)], idx_v)
        for k in range(_NCHUNK):
            pltpu.async_copy(tab.at[idx_v.at[pl.ds(k * _CHUNK, _CHUNK)]],
                             rows_v, sem).wait()
            pltpu.sync_copy(rows_v,
                            out.at[pl.ds(base + k * _CHUNK, _CHUNK)])


def kernel(values_f1, lengths_f1, values_f2, lengths_f2,
           values_f3, lengths_f3, values_f4, lengths_f4,
           table_f1, table_f2, table_f3, table_f4):
    tab = jnp.concatenate([table_f1, table_f2, table_f3, table_f4], axis=1)
    o1, o2, o3, o4 = _gather4(values_f1, values_f2, values_f3, values_f4, tab)
    return (o1[:, 0:DIM], lengths_f1,
            o2[:, DIM:2 * DIM], lengths_f2,
            o3[:, 2 * DIM:3 * DIM], lengths_f3,
            o4[:, 3 * DIM:4 * DIM], lengths_f4)
